# baseline (speedup 1.0000x reference)
import functools

import jax
import jax.numpy as jnp
from jax import lax
from jax.experimental import pallas as pl
from jax.experimental.pallas import tpu as pltpu
from jax.experimental.pallas import tpu_sc as plsc

HIDDEN = 64
CHUNK = 128


def _trig_body(rel_ref, cos_ref, sin_ref):
    theta = rel_ref[...]
    cos_ref[...] = jnp.cos(theta)
    sin_ref[...] = jnp.sin(theta)


def _trig_tables(rel_emb):
    r, d = rel_emb.shape
    out = pl.pallas_call(
        _trig_body,
        out_shape=(
            jax.ShapeDtypeStruct((r, d), rel_emb.dtype),
            jax.ShapeDtypeStruct((r, d), rel_emb.dtype),
        ),
    )(rel_emb)
    return out


def _make_sc_gather(batch, d, nw):
    b_per_w = batch // nw
    n_chunks = b_per_w // CHUNK
    mesh = plsc.VectorSubcoreMesh(core_axis_name="c", subcore_axis_name="s")
    out_sds = jax.ShapeDtypeStruct((nw, n_chunks, CHUNK, d), jnp.float32)

    @functools.partial(
        pl.kernel,
        mesh=mesh,
        compiler_params=pltpu.CompilerParams(use_tc_tiling_on_sc=False),
        out_type=tuple(out_sds for _ in range(6)),
        scratch_types=[
            pltpu.VMEM((n_chunks, CHUNK), jnp.int32),
            pltpu.VMEM((n_chunks, CHUNK), jnp.int32),
            pltpu.VMEM((n_chunks, CHUNK), jnp.int32),
            pltpu.VMEM((CHUNK, d), jnp.float32),
            pltpu.VMEM((CHUNK, d), jnp.float32),
            pltpu.SemaphoreType.DMA,
            pltpu.SemaphoreType.DMA,
            pltpu.SemaphoreType.DMA,
            pltpu.SemaphoreType.DMA,
        ],
    )
    def sc_gather(h_idx, r_idx, t_idx, t_re, t_im, t_cos, t_sin,
                  o_hre, o_him, o_rre, o_rim, o_tre, o_tim,
                  hv, rv, tv, buf0, buf1, sg0, sg1, sw0, sw1):
        nc = 2
        wid = lax.axis_index("s") * nc + lax.axis_index("c")
        pltpu.sync_copy(h_idx.at[wid], hv)
        pltpu.sync_copy(r_idx.at[wid], rv)
        pltpu.sync_copy(t_idx.at[wid], tv)

        bufs = (buf0, buf1)
        sgs = (sg0, sg1)
        sws = (sw0, sw1)
        pairs = [
            (t_re, hv, o_hre),
            (t_im, hv, o_him),
            (t_cos, rv, o_rre),
            (t_sin, rv, o_rim),
            (t_re, tv, o_tre),
            (t_im, tv, o_tim),
        ]
        items = [(tab, idx, out, j)
                 for (tab, idx, out) in pairs for j in range(n_chunks)]
        n = len(items)
        gather_waits = [None] * n
        write_waits = [None] * n

        def start_gather(i):
            tab, idx, _out, j = items[i]
            b = bufs[i % 2]
            gather_waits[i] = pltpu.async_copy(tab.at[idx.at[j]], b, sgs[i % 2])

        def start_write(i):
            _tab, _idx, out, j = items[i]
            b = bufs[i % 2]
            write_waits[i] = pltpu.async_copy(b, out.at[wid, j], sws[i % 2])

        start_gather(0)
        for i in range(n):
            if i + 1 < n:
                if i >= 1:
                    write_waits[i - 1].wait()
                start_gather(i + 1)
            gather_waits[i].wait()
            start_write(i)
        write_waits[n - 2].wait()
        write_waits[n - 1].wait()

    return sc_gather, b_per_w, n_chunks


def kernel(head_index, rel_type, tail_index, node_emb, node_emb_im, rel_emb):
    batch = head_index.shape[0]
    d = node_emb.shape[1]
    info = plsc.get_sparse_core_info()
    nw = info.num_cores * info.num_subcores
    b_per_w = batch // nw
    n_chunks = b_per_w // CHUNK

    rel_cos, rel_sin = _trig_tables(rel_emb)

    h_idx = head_index.astype(jnp.int32).reshape(nw, n_chunks, CHUNK)
    r_idx = rel_type.astype(jnp.int32).reshape(nw, n_chunks, CHUNK)
    t_idx = tail_index.astype(jnp.int32).reshape(nw, n_chunks, CHUNK)

    sc_gather, _, _ = _make_sc_gather(batch, d, nw)
    outs = sc_gather(h_idx, r_idx, t_idx, node_emb, node_emb_im,
                     rel_cos, rel_sin)
    return tuple(o.reshape(batch, d) for o in outs)
